# baseline (speedup 1.0000x reference)
import functools

import jax
import jax.numpy as jnp
from jax import lax
from jax.experimental import pallas as pl
from jax.experimental.pallas import tpu as pltpu

_NH = 2
_LOG2E = 1.4426950408889634


def _gat_kernel(adj_ref, x_ref, w_ref, w_gcn_ref,
                s_ref, adj8_ref, z_scr, ad_scr, ast_scr):
    f32 = jnp.float32
    hd = w_gcn_ref.shape[0] // _NH
    fe = x_ref.shape[1]
    i = pl.program_id(0)

    @pl.when(i == 0)
    def _prep():
        wtop = w_ref[0:fe, :]
        w_gat = w_ref[fe:fe + fe, :]
        tail = w_ref[fe + fe:fe + fe + 8, :]
        w_zf = jnp.dot(wtop[:, 0:fe], w_gat, preferred_element_type=f32)
        b_zf = jnp.dot(tail[0:1, 0:fe], w_gat, preferred_element_type=f32)
        z = jnp.dot(x_ref[...], w_zf, preferred_element_type=f32) + b_zf
        z_scr[:, 0:hd] = z[:, 0:hd].astype(jnp.bfloat16)
        z_scr[:, hd:hd + 128] = jnp.ones((z.shape[0], 128), jnp.bfloat16)
        z_scr[:, hd + 128:] = z[:, hd:2 * hd].astype(jnp.bfloat16)
        for (a0, ref) in ((fe + 2, None), (fe, ast_scr)):
            c = [jnp.dot(z[:, h * hd:(h + 1) * hd],
                         wtop[:, a0 + h:a0 + h + 1] * _LOG2E,
                         preferred_element_type=f32) for h in range(_NH)]
            cols = jnp.concatenate(c + [0.2 * c[0], 0.2 * c[1]], axis=1)
            if ref is None:
                ad_scr[...] = cols
            else:
                ref[...] = jnp.swapaxes(cols, 0, 1)

    b_gat_ref = w_ref[2 * fe:2 * fe + 8, :][1:2, :]

    adj = adj_ref[...]
    adj8_ref[...] = jnp.round(adj).astype(jnp.int8).astype(jnp.int4)
    adj_bf = adj.astype(jnp.bfloat16)
    deg = jnp.sum(adj, axis=1, keepdims=True)
    dinv = lax.rsqrt(deg)
    tm = adj.shape[0]
    out = jnp.zeros((tm, s_ref.shape[1]), f32)
    for h in range(_NH):
        ad = ad_scr[pl.ds(i * tm, tm), :]
        e = jnp.maximum(ad[:, h:h + 1] + ast_scr[h:h + 1, :],
                        ad[:, 2 + h:3 + h] + ast_scr[2 + h:3 + h, :])
        p = jnp.exp2(e).astype(jnp.bfloat16) * adj_bf
        ext = jnp.dot(p, z_scr[:, h * hd:h * hd + hd + 128],
                      preferred_element_type=f32)
        if h == 0:
            acc, l = ext[:, 0:hd], ext[:, hd:hd + 1]
        else:
            acc, l = ext[:, 128:128 + hd], ext[:, 0:1]
        g = jnp.maximum(acc / l + b_gat_ref[:, h * hd:(h + 1) * hd], 0.0)
        out = out + jnp.dot(g.astype(jnp.bfloat16),
                            w_gcn_ref[h * hd:(h + 1) * hd, :],
                            preferred_element_type=f32)
    s_ref[...] = (out * dinv).astype(jnp.bfloat16)


def _gcn_kernel(adj8_ref, s_ref, b_gcn_ref, out_ref):
    f32 = jnp.float32
    adj_bf = adj8_ref[...].astype(jnp.int8).astype(jnp.bfloat16)
    deg = jnp.sum(adj_bf.astype(f32), axis=1, keepdims=True)
    acc = jnp.dot(adj_bf, s_ref[...], preferred_element_type=f32)
    out_ref[...] = jnp.maximum(
        acc * lax.rsqrt(deg) + b_gcn_ref[...], 0.0)


def _full(a):
    zeros = (0,) * a.ndim
    return pl.BlockSpec(a.shape, lambda i, zeros=zeros: zeros)


@jax.jit
def _run(x, adj, w_emb, b_emb, w_gat, att_src, att_dst, b_gat, w_gcn, b_gcn):
    f32 = jnp.float32
    n, f_in = x.shape
    h_hd = w_gat.shape[1]
    hid = w_gcn.shape[1]

    x = x.astype(f32)
    adj = adj.astype(f32)
    w_gcn_bf = w_gcn.astype(jnp.bfloat16)
    b_gcn = b_gcn.astype(f32)

    wtop = jnp.concatenate(
        [w_emb.astype(f32), att_src.astype(f32), att_dst.astype(f32),
         jnp.zeros((f_in, h_hd - f_in - 2 * _NH), f32)], axis=1)
    tail = jnp.concatenate(
        [jnp.concatenate([b_emb.astype(f32),
                          jnp.zeros((1, h_hd - f_in), f32)], axis=1),
         b_gat.astype(f32), jnp.zeros((6, h_hd), f32)], axis=0)
    w_pack = jnp.concatenate([wtop, w_gat.astype(f32), tail], axis=0)

    tm = 512 if n % 512 == 0 else n
    tg = 2048 if n % 2048 == 0 else n
    zw = h_hd + 128

    cp = pltpu.CompilerParams(
        dimension_semantics=("arbitrary",),
        vmem_limit_bytes=64 * 1024 * 1024)

    s, adj8 = pl.pallas_call(
        _gat_kernel,
        out_shape=(jax.ShapeDtypeStruct((n, hid), jnp.bfloat16),
                   jax.ShapeDtypeStruct((n, n), jnp.int4)),
        grid=(n // tm,),
        in_specs=[pl.BlockSpec((tm, n), lambda i: (i, 0)),
                  _full(x), _full(w_pack), _full(w_gcn_bf)],
        out_specs=[pl.BlockSpec((tm, hid), lambda i: (i, 0)),
                   pl.BlockSpec((tm, n), lambda i: (i, 0))],
        scratch_shapes=[pltpu.VMEM((n, zw), jnp.bfloat16),
                        pltpu.VMEM((n, 4), f32),
                        pltpu.VMEM((4, n), f32)],
        compiler_params=cp,
    )(adj, x, w_pack, w_gcn_bf)

    out = pl.pallas_call(
        _gcn_kernel,
        out_shape=jax.ShapeDtypeStruct((n, hid), f32),
        grid=(n // tg,),
        in_specs=[pl.BlockSpec((tg, n), lambda i: (i, 0)),
                  _full(s), _full(b_gcn)],
        out_specs=pl.BlockSpec((tg, hid), lambda i: (i, 0)),
        compiler_params=cp,
    )(adj8, s, b_gcn)
    return out


def kernel(x, adj, w_emb, b_emb, w_gat, att_src, att_dst, b_gat, w_gcn, b_gcn):
    return _run(x, adj, w_emb, b_emb, w_gat, att_src, att_dst,
                b_gat, w_gcn, b_gcn)


# speedup vs baseline: 1.0374x; 1.0374x over previous
import functools

import jax
import jax.numpy as jnp
from jax import lax
from jax.experimental import pallas as pl
from jax.experimental.pallas import tpu as pltpu

_NH = 2
_LOG2E = 1.4426950408889634


def _gat_kernel(adj_ref, x_ref, w_emb_ref, b_emb_ref, w_gat_ref,
                as_ref, adst_ref, b_gat_ref, w_gcn_ref,
                s_ref, adj8_ref, z_scr, ad_scr, ast_scr):
    f32 = jnp.float32
    hd = w_gcn_ref.shape[0] // _NH
    i = pl.program_id(0)

    @pl.when(i == 0)
    def _prep():
        w_zf = jnp.dot(w_emb_ref[...], w_gat_ref[...],
                       preferred_element_type=f32)
        b_zf = jnp.dot(b_emb_ref[...], w_gat_ref[...],
                       preferred_element_type=f32)
        z = jnp.dot(x_ref[...], w_zf, preferred_element_type=f32) + b_zf
        z_scr[:, 0:hd] = z[:, 0:hd].astype(jnp.bfloat16)
        z_scr[:, hd:hd + 128] = jnp.ones((z.shape[0], 128), jnp.bfloat16)
        z_scr[:, hd + 128:] = z[:, hd:2 * hd].astype(jnp.bfloat16)
        for (att, ref) in ((adst_ref, None), (as_ref, ast_scr)):
            c = [jnp.dot(z[:, h * hd:(h + 1) * hd], att[:, h:h + 1] * _LOG2E,
                         preferred_element_type=f32) for h in range(_NH)]
            cols = jnp.concatenate(c + [0.2 * c[0], 0.2 * c[1]], axis=1)
            if ref is None:
                ad_scr[...] = cols
            else:
                ref[...] = jnp.swapaxes(cols, 0, 1)

    adj = adj_ref[...]
    adj8_ref[...] = jnp.round(adj).astype(jnp.int8).astype(jnp.int4)
    adj_bf = adj.astype(jnp.bfloat16)
    deg = jnp.sum(adj, axis=1, keepdims=True)
    dinv = lax.rsqrt(deg)
    tm = adj.shape[0]
    out = jnp.zeros((tm, s_ref.shape[1]), f32)
    for h in range(_NH):
        ad = ad_scr[pl.ds(i * tm, tm), :]
        e = jnp.maximum(ad[:, h:h + 1] + ast_scr[h:h + 1, :],
                        ad[:, 2 + h:3 + h] + ast_scr[2 + h:3 + h, :])
        p = jnp.exp2(e).astype(jnp.bfloat16) * adj_bf
        ext = jnp.dot(p, z_scr[:, h * hd:h * hd + hd + 128],
                      preferred_element_type=f32)
        if h == 0:
            acc, l = ext[:, 0:hd], ext[:, hd:hd + 1]
        else:
            acc, l = ext[:, 128:128 + hd], ext[:, 0:1]
        g = jnp.maximum(acc / l + b_gat_ref[:, h * hd:(h + 1) * hd], 0.0)
        out = out + jnp.dot(g.astype(jnp.bfloat16),
                            w_gcn_ref[h * hd:(h + 1) * hd, :],
                            preferred_element_type=f32)
    s_ref[...] = (out * dinv).astype(jnp.bfloat16)


def _gcn_kernel(adj8_ref, s_ref, b_gcn_ref, out_ref):
    f32 = jnp.float32
    adj_bf = adj8_ref[...].astype(jnp.int8).astype(jnp.bfloat16)
    deg = jnp.sum(adj_bf.astype(f32), axis=1, keepdims=True)
    acc = jnp.dot(adj_bf, s_ref[...], preferred_element_type=f32)
    out_ref[...] = jnp.maximum(
        acc * lax.rsqrt(deg) + b_gcn_ref[...], 0.0)


def _full(a):
    zeros = (0,) * a.ndim
    return pl.BlockSpec(a.shape, lambda i, zeros=zeros: zeros)


@jax.jit
def _run(x, adj, w_emb, b_emb, w_gat, att_src, att_dst, b_gat, w_gcn, b_gcn):
    f32 = jnp.float32
    n, f_in = x.shape
    h_hd = w_gat.shape[1]
    hid = w_gcn.shape[1]

    x = x.astype(f32)
    adj = adj.astype(f32)
    w_emb = w_emb.astype(f32)
    b_emb = b_emb.astype(f32)
    w_gat = w_gat.astype(f32)
    att_src = att_src.astype(f32)
    att_dst = att_dst.astype(f32)
    b_gat = b_gat.astype(f32)
    w_gcn_bf = w_gcn.astype(jnp.bfloat16)
    b_gcn = b_gcn.astype(f32)

    tm = 512 if n % 512 == 0 else n
    tg = 4096 if n % 4096 == 0 else n
    zw = h_hd + 128

    cp = pltpu.CompilerParams(
        dimension_semantics=("arbitrary",),
        vmem_limit_bytes=64 * 1024 * 1024)

    s, adj8 = pl.pallas_call(
        _gat_kernel,
        out_shape=(jax.ShapeDtypeStruct((n, hid), jnp.bfloat16),
                   jax.ShapeDtypeStruct((n, n), jnp.int4)),
        grid=(n // tm,),
        in_specs=[pl.BlockSpec((tm, n), lambda i: (i, 0)),
                  _full(x), _full(w_emb), _full(b_emb), _full(w_gat),
                  _full(att_src), _full(att_dst),
                  _full(b_gat), _full(w_gcn_bf)],
        out_specs=[pl.BlockSpec((tm, hid), lambda i: (i, 0)),
                   pl.BlockSpec((tm, n), lambda i: (i, 0))],
        scratch_shapes=[pltpu.VMEM((n, zw), jnp.bfloat16),
                        pltpu.VMEM((n, 4), f32),
                        pltpu.VMEM((4, n), f32)],
        compiler_params=cp,
    )(adj, x, w_emb, b_emb, w_gat, att_src, att_dst, b_gat, w_gcn_bf)

    out = pl.pallas_call(
        _gcn_kernel,
        out_shape=jax.ShapeDtypeStruct((n, hid), f32),
        grid=(n // tg,),
        in_specs=[pl.BlockSpec((tg, n), lambda i: (i, 0)),
                  _full(s), _full(b_gcn)],
        out_specs=pl.BlockSpec((tg, hid), lambda i: (i, 0)),
        compiler_params=cp,
    )(adj8, s, b_gcn)
    return out


def kernel(x, adj, w_emb, b_emb, w_gat, att_src, att_dst, b_gat, w_gcn, b_gcn):
    return _run(x, adj, w_emb, b_emb, w_gat, att_src, att_dst,
                b_gat, w_gcn, b_gcn)


# speedup vs baseline: 1.0731x; 1.0344x over previous
import functools

import jax
import jax.numpy as jnp
from jax import lax
from jax.experimental import pallas as pl
from jax.experimental.pallas import tpu as pltpu

_NH = 2
_LOG2E = 1.4426950408889634


def _gat_kernel(adj_ref, x_ref, w_emb_ref, b_emb_ref, w_gat_ref,
                as_ref, adst_ref, b_gat_ref, w_gcn_ref,
                s_ref, adj8_ref, z_scr, ad_scr, ast_scr):
    f32 = jnp.float32
    hd = w_gcn_ref.shape[0] // _NH
    i = pl.program_id(0)

    @pl.when(i == 0)
    def _prep():
        w_zf = jnp.dot(w_emb_ref[...], w_gat_ref[...],
                       preferred_element_type=f32)
        b_zf = jnp.dot(b_emb_ref[...], w_gat_ref[...],
                       preferred_element_type=f32)
        z = jnp.dot(x_ref[...], w_zf, preferred_element_type=f32) + b_zf
        z_scr[:, 0:hd] = z[:, 0:hd].astype(jnp.bfloat16)
        z_scr[:, hd:hd + 128] = jnp.ones((z.shape[0], 128), jnp.bfloat16)
        z_scr[:, hd + 128:] = z[:, hd:2 * hd].astype(jnp.bfloat16)
        for (att, ref) in ((adst_ref, None), (as_ref, ast_scr)):
            c = [jnp.dot(z[:, h * hd:(h + 1) * hd], att[:, h:h + 1] * _LOG2E,
                         preferred_element_type=f32) for h in range(_NH)]
            cols = jnp.concatenate(c + [0.2 * c[0], 0.2 * c[1]], axis=1)
            if ref is None:
                ad_scr[...] = cols
            else:
                ref[...] = jnp.swapaxes(cols, 0, 1)

    adj = adj_ref[...]
    adj8_ref[...] = jnp.round(adj).astype(jnp.int8).astype(jnp.int4)
    adj_bf = adj.astype(jnp.bfloat16)
    deg = jnp.sum(adj, axis=1, keepdims=True)
    dinv = lax.rsqrt(deg)
    tm = adj.shape[0]
    out = jnp.zeros((tm, s_ref.shape[1]), f32)
    for h in range(_NH):
        ad = ad_scr[pl.ds(i * tm, tm), :]
        e = jnp.maximum(ad[:, h:h + 1] + ast_scr[h:h + 1, :],
                        ad[:, 2 + h:3 + h] + ast_scr[2 + h:3 + h, :])
        p = jnp.exp2(e).astype(jnp.bfloat16) * adj_bf
        ext = jnp.dot(p, z_scr[:, h * hd:h * hd + hd + 128],
                      preferred_element_type=f32)
        if h == 0:
            acc, l = ext[:, 0:hd], ext[:, hd:hd + 1]
        else:
            acc, l = ext[:, 128:128 + hd], ext[:, 0:1]
        g = jnp.maximum(acc / l + b_gat_ref[:, h * hd:(h + 1) * hd], 0.0)
        out = out + jnp.dot(g.astype(jnp.bfloat16),
                            w_gcn_ref[h * hd:(h + 1) * hd, :],
                            preferred_element_type=f32)
    s_ref[...] = (out * dinv).astype(jnp.bfloat16)


def _gcn_kernel(adj8_ref, s_ref, b_gcn_ref, out_ref):
    f32 = jnp.float32
    adj_bf = adj8_ref[...].astype(jnp.int8).astype(jnp.bfloat16)
    deg = jnp.sum(adj_bf.astype(f32), axis=1, keepdims=True)
    acc = jnp.dot(adj_bf, s_ref[...], preferred_element_type=f32)
    out_ref[...] = jnp.maximum(
        acc * lax.rsqrt(deg) + b_gcn_ref[...], 0.0)


def _full(a):
    zeros = (0,) * a.ndim
    return pl.BlockSpec(a.shape, lambda i, zeros=zeros: zeros)


@jax.jit
def _run(x, adj, w_emb, b_emb, w_gat, att_src, att_dst, b_gat, w_gcn, b_gcn):
    f32 = jnp.float32
    n, f_in = x.shape
    h_hd = w_gat.shape[1]
    hid = w_gcn.shape[1]

    x = x.astype(f32)
    adj = adj.astype(f32)
    w_emb = w_emb.astype(f32)
    b_emb = b_emb.astype(f32)
    w_gat = w_gat.astype(f32)
    att_src = att_src.astype(f32)
    att_dst = att_dst.astype(f32)
    b_gat = b_gat.astype(f32)
    w_gcn_bf = w_gcn.astype(jnp.bfloat16)
    b_gcn = b_gcn.astype(f32)

    tm = 512 if n % 512 == 0 else n
    tg = 1024 if n % 1024 == 0 else n
    zw = h_hd + 128

    cp = pltpu.CompilerParams(
        dimension_semantics=("arbitrary",),
        vmem_limit_bytes=64 * 1024 * 1024)

    s, adj8 = pl.pallas_call(
        _gat_kernel,
        out_shape=(jax.ShapeDtypeStruct((n, hid), jnp.bfloat16),
                   jax.ShapeDtypeStruct((n, n), jnp.int4)),
        grid=(n // tm,),
        in_specs=[pl.BlockSpec((tm, n), lambda i: (i, 0)),
                  _full(x), _full(w_emb), _full(b_emb), _full(w_gat),
                  _full(att_src), _full(att_dst),
                  _full(b_gat), _full(w_gcn_bf)],
        out_specs=[pl.BlockSpec((tm, hid), lambda i: (i, 0)),
                   pl.BlockSpec((tm, n), lambda i: (i, 0))],
        scratch_shapes=[pltpu.VMEM((n, zw), jnp.bfloat16),
                        pltpu.VMEM((n, 4), f32),
                        pltpu.VMEM((4, n), f32)],
        compiler_params=cp,
    )(adj, x, w_emb, b_emb, w_gat, att_src, att_dst, b_gat, w_gcn_bf)

    out = pl.pallas_call(
        _gcn_kernel,
        out_shape=jax.ShapeDtypeStruct((n, hid), f32),
        grid=(n // tg,),
        in_specs=[pl.BlockSpec((tg, n), lambda i: (i, 0)),
                  _full(s), _full(b_gcn)],
        out_specs=pl.BlockSpec((tg, hid), lambda i: (i, 0)),
        compiler_params=cp,
    )(adj8, s, b_gcn)
    return out


def kernel(x, adj, w_emb, b_emb, w_gat, att_src, att_dst, b_gat, w_gcn, b_gcn):
    return _run(x, adj, w_emb, b_emb, w_gat, att_src, att_dst,
                b_gat, w_gcn, b_gcn)


# speedup vs baseline: 1.0773x; 1.0039x over previous
import functools

import jax
import jax.numpy as jnp
from jax import lax
from jax.experimental import pallas as pl
from jax.experimental.pallas import tpu as pltpu

_NH = 2
_LOG2E = 1.4426950408889634


def _gat_kernel(adj_ref, x_ref, w_emb_ref, b_emb_ref, w_gat_ref,
                as_ref, adst_ref, b_gat_ref, w_gcn_ref,
                s_ref, adj8_ref, z_scr, ad_scr, ast_scr):
    f32 = jnp.float32
    hd = w_gcn_ref.shape[0] // _NH
    i = pl.program_id(0)

    @pl.when(i == 0)
    def _prep():
        w_zf = jnp.dot(w_emb_ref[...], w_gat_ref[...],
                       preferred_element_type=f32)
        b_zf = jnp.dot(b_emb_ref[...], w_gat_ref[...],
                       preferred_element_type=f32)
        z = jnp.dot(x_ref[...], w_zf, preferred_element_type=f32) + b_zf
        z_scr[:, 0:hd] = z[:, 0:hd].astype(jnp.bfloat16)
        z_scr[:, hd:hd + 128] = jnp.ones((z.shape[0], 128), jnp.bfloat16)
        z_scr[:, hd + 128:] = z[:, hd:2 * hd].astype(jnp.bfloat16)
        for (att, ref) in ((adst_ref, None), (as_ref, ast_scr)):
            c = [jnp.dot(z[:, h * hd:(h + 1) * hd], att[:, h:h + 1] * _LOG2E,
                         preferred_element_type=f32) for h in range(_NH)]
            cols = jnp.concatenate(c + [0.2 * c[0], 0.2 * c[1]], axis=1)
            if ref is None:
                ad_scr[...] = cols
            else:
                ref[...] = jnp.swapaxes(cols, 0, 1)

    adj = adj_ref[...]
    adj8_ref[...] = jnp.round(adj).astype(jnp.int8).astype(jnp.int4)
    adj_bf = adj.astype(jnp.bfloat16)
    deg = jnp.sum(adj, axis=1, keepdims=True)
    dinv = lax.rsqrt(deg)
    tm = adj.shape[0]
    out = jnp.zeros((tm, s_ref.shape[1]), f32)
    for h in range(_NH):
        ad = ad_scr[pl.ds(i * tm, tm), :]
        e = jnp.maximum(ad[:, h:h + 1] + ast_scr[h:h + 1, :],
                        ad[:, 2 + h:3 + h] + ast_scr[2 + h:3 + h, :])
        p = jnp.exp2(e).astype(jnp.bfloat16) * adj_bf
        ext = jnp.dot(p, z_scr[:, h * hd:h * hd + hd + 128],
                      preferred_element_type=f32)
        if h == 0:
            acc, l = ext[:, 0:hd], ext[:, hd:hd + 1]
        else:
            acc, l = ext[:, 128:128 + hd], ext[:, 0:1]
        g = jnp.maximum(acc / l + b_gat_ref[:, h * hd:(h + 1) * hd], 0.0)
        out = out + jnp.dot(g.astype(jnp.bfloat16),
                            w_gcn_ref[h * hd:(h + 1) * hd, :],
                            preferred_element_type=f32)
    s_ref[...] = (out * dinv).astype(jnp.bfloat16)


def _gcn_kernel(adj8_ref, s_ref, b_gcn_ref, out_ref):
    f32 = jnp.float32
    adj_bf = adj8_ref[...].astype(jnp.int8).astype(jnp.bfloat16)
    deg = jnp.sum(adj_bf.astype(f32), axis=1, keepdims=True)
    acc = jnp.dot(adj_bf, s_ref[...], preferred_element_type=f32)
    out_ref[...] = jnp.maximum(
        acc * lax.rsqrt(deg) + b_gcn_ref[...], 0.0)


def _full(a):
    zeros = (0,) * a.ndim
    return pl.BlockSpec(a.shape, lambda i, zeros=zeros: zeros)


@jax.jit
def _run(x, adj, w_emb, b_emb, w_gat, att_src, att_dst, b_gat, w_gcn, b_gcn):
    f32 = jnp.float32
    n, f_in = x.shape
    h_hd = w_gat.shape[1]
    hid = w_gcn.shape[1]

    x = x.astype(f32)
    adj = adj.astype(f32)
    w_emb = w_emb.astype(f32)
    b_emb = b_emb.astype(f32)
    w_gat = w_gat.astype(f32)
    att_src = att_src.astype(f32)
    att_dst = att_dst.astype(f32)
    b_gat = b_gat.astype(f32)
    w_gcn_bf = w_gcn.astype(jnp.bfloat16)
    b_gcn = b_gcn.astype(f32)

    tm = 512 if n % 512 == 0 else n
    tg = 512 if n % 512 == 0 else n
    zw = h_hd + 128

    cp = pltpu.CompilerParams(
        dimension_semantics=("arbitrary",),
        vmem_limit_bytes=64 * 1024 * 1024)

    s, adj8 = pl.pallas_call(
        _gat_kernel,
        out_shape=(jax.ShapeDtypeStruct((n, hid), jnp.bfloat16),
                   jax.ShapeDtypeStruct((n, n), jnp.int4)),
        grid=(n // tm,),
        in_specs=[pl.BlockSpec((tm, n), lambda i: (i, 0)),
                  _full(x), _full(w_emb), _full(b_emb), _full(w_gat),
                  _full(att_src), _full(att_dst),
                  _full(b_gat), _full(w_gcn_bf)],
        out_specs=[pl.BlockSpec((tm, hid), lambda i: (i, 0)),
                   pl.BlockSpec((tm, n), lambda i: (i, 0))],
        scratch_shapes=[pltpu.VMEM((n, zw), jnp.bfloat16),
                        pltpu.VMEM((n, 4), f32),
                        pltpu.VMEM((4, n), f32)],
        compiler_params=cp,
    )(adj, x, w_emb, b_emb, w_gat, att_src, att_dst, b_gat, w_gcn_bf)

    out = pl.pallas_call(
        _gcn_kernel,
        out_shape=jax.ShapeDtypeStruct((n, hid), f32),
        grid=(n // tg,),
        in_specs=[pl.BlockSpec((tg, n), lambda i: (i, 0)),
                  _full(s), _full(b_gcn)],
        out_specs=pl.BlockSpec((tg, hid), lambda i: (i, 0)),
        compiler_params=cp,
    )(adj8, s, b_gcn)
    return out


def kernel(x, adj, w_emb, b_emb, w_gat, att_src, att_dst, b_gat, w_gcn, b_gcn):
    return _run(x, adj, w_emb, b_emb, w_gat, att_src, att_dst,
                b_gat, w_gcn, b_gcn)
